# baseline (speedup 1.0000x reference)
import functools

import jax
import jax.numpy as jnp
from jax import lax
from jax.experimental import pallas as pl
from jax.experimental.pallas import tpu as pltpu
from jax.experimental.pallas import tpu_sc as plsc

N = 1000000
D = 64
B = 16384

NC = 2
NS = 16
NW = NC * NS
B_PER_W = B // NW

_mesh = plsc.VectorSubcoreMesh(core_axis_name="c", subcore_axis_name="s")


@functools.partial(
    pl.kernel,
    mesh=_mesh,
    compiler_params=pltpu.CompilerParams(use_tc_tiling_on_sc=False),
    out_type=jax.ShapeDtypeStruct((B, D), jnp.float32),
    scratch_types=[
        pltpu.VMEM((B_PER_W,), jnp.int32),
        pltpu.VMEM((B_PER_W, D), jnp.float32),
        pltpu.SemaphoreType.DMA,
    ],
)
def _gather_kernel(table_hbm, idx_hbm, out_hbm, idx_v, rows_v, sem):
    wid = lax.axis_index("s") * NC + lax.axis_index("c")
    base = wid * B_PER_W
    pltpu.sync_copy(idx_hbm.at[pl.ds(base, B_PER_W)], idx_v)
    pltpu.async_copy(table_hbm.at[idx_v], rows_v, sem).wait()
    pltpu.sync_copy(rows_v, out_hbm.at[pl.ds(base, B_PER_W)])


def kernel(nodes, table):
    return _gather_kernel(table, nodes.astype(jnp.int32))


# speedup vs baseline: 1.2846x; 1.2846x over previous
import functools

import jax
import jax.numpy as jnp
from jax import lax
from jax.experimental import pallas as pl
from jax.experimental.pallas import tpu as pltpu
from jax.experimental.pallas import tpu_sc as plsc

N = 1000000
D = 64
B = 16384

NC = 2
NS = 16
NW = NC * NS
B_PER_W = B // NW
R_PER_W = B_PER_W * D // 128

_mesh = plsc.VectorSubcoreMesh(core_axis_name="c", subcore_axis_name="s")


@functools.partial(
    pl.kernel,
    mesh=_mesh,
    out_type=jax.ShapeDtypeStruct((B * D // 128, 128), jnp.float32),
    scratch_types=[
        pltpu.VMEM((B_PER_W,), jnp.int32),
        pltpu.SemaphoreType.DMA,
    ],
)
def _gather_kernel(table_hbm, idx_hbm, out_hbm, idx_v, sem):
    wid = lax.axis_index("s") * NC + lax.axis_index("c")
    base = wid * B_PER_W
    obase = wid * R_PER_W
    pltpu.sync_copy(idx_hbm.at[pl.ds(base, B_PER_W)], idx_v)

    def body(g, carry):
        vec = idx_v[pl.ds(g * 16, 16)]
        for u in range(16):
            dst = out_hbm.at[obase + g * 8 + u // 2, pl.ds((u % 2) * D, D)]
            pltpu.async_copy(table_hbm.at[vec[u]], dst, sem)
        return carry

    lax.fori_loop(0, B_PER_W // 16, body, 0)
    pltpu.make_async_copy(
        table_hbm.at[pl.ds(0, R_PER_W * 2)],
        out_hbm.at[pl.ds(obase, R_PER_W)],
        sem,
    ).wait()


def kernel(nodes, table):
    out = _gather_kernel(table, nodes.astype(jnp.int32))
    return out.reshape(B, D)
